# Initial kernel scaffold; baseline (speedup 1.0000x reference)
import jax
import jax.numpy as jnp
from jax.experimental import pallas as pl


def kernel(x, edge_index, batch, global_features, emb, W1, b1, ln1_g, ln1_b, W2, b2, ln2_g, ln2_b, gW1, gb1, gW2, gb2, cW1, cb1, cW2, cb2):
    raise NotImplementedError("write your pallas kernel here")


# speedup vs baseline: 7.3633x; 7.3633x over previous
import functools

import jax
import jax.numpy as jnp
from jax import lax
from jax.experimental import pallas as pl
from jax.experimental.pallas import tpu as pltpu
from jax.experimental.pallas import tpu_sc as plsc

_N = 10000
_E = 320000
_F_IN = 128
_EMB = 16
_GDIM = 16
_NG = 64
_H1 = 256
_H2 = 256
_PH = 128

_NP = 10240
_NB = 512
_NSC = 2
_NT = 16
_KB = 128
_EP = 323584
_STEPS = _EP // _NT // _KB
_DSTEPS = _EP // (_NT * _NSC) // _KB
_EPT = _KB * _STEPS
_ACC = 10112
_RPT = _ACC // _NT


def _sc_deg_emb(dst_pad, types_pad, emb):
    mesh = plsc.VectorSubcoreMesh(core_axis_name="c", subcore_axis_name="s")

    @functools.partial(
        pl.kernel,
        out_type=[
            jax.ShapeDtypeStruct((_NSC, _NP, 128), jnp.float32),
            jax.ShapeDtypeStruct((_NP, 128), jnp.float32),
        ],
        mesh=mesh,
        scratch_types=[
            pltpu.VMEM((1, _KB), jnp.int32),
            pltpu.VMEM((1, 64), jnp.int32),
            pltpu.VMEM((_KB, 128), jnp.float32),
            pltpu.VMEM((64, 128), jnp.float32),
            pltpu.VMEM_SHARED((_ACC, 128), jnp.float32),
            pltpu.SemaphoreType.DMA,
        ],
    )
    def k(dst_hbm, types_hbm, emb_hbm, deg_out, embx_out, idxb, gidxb,
          onesb, rowsb, acc, sem):
        c = lax.axis_index("c")
        s = lax.axis_index("s")
        wid = s * _NSC + c

        def zr(i, _):
            def zq(q, _):
                onesb[i, pl.ds(q * 16, 16)] = jnp.zeros((16,), jnp.float32)
                return 0

            lax.fori_loop(0, 8, zq, 0)
            return 0

        lax.fori_loop(0, _KB, zr, 0)
        for j in range(4):
            pltpu.sync_copy(onesb, acc.at[pl.ds(s * _RPT + j * _KB, _KB)])
        pltpu.sync_copy(onesb.at[pl.ds(0, _RPT - 4 * _KB)],
                        acc.at[pl.ds(s * _RPT + 4 * _KB, _RPT - 4 * _KB)])

        def orow(i, _):
            def oq(q, _):
                onesb[i, pl.ds(q * 16, 16)] = jnp.ones((16,), jnp.float32)
                return 0

            lax.fori_loop(0, 8, oq, 0)
            return 0

        lax.fori_loop(0, _KB, orow, 0)
        plsc.subcore_barrier()

        dbase = wid * _DSTEPS * _KB

        def body(b, _):
            off = dbase + b * _KB
            pltpu.sync_copy(dst_hbm.at[pl.ds(off, _KB)], idxb.at[0])
            pltpu.sync_copy(onesb, acc.at[idxb.at[0]], add=True)
            return 0

        lax.fori_loop(0, _DSTEPS, body, 0)
        plsc.subcore_barrier()
        pltpu.sync_copy(acc.at[pl.ds(s * _RPT, _RPT)],
                        deg_out.at[c, pl.ds(s * _RPT, _RPT)])

        gbase = wid * (_NP // _NT // _NSC)

        def gbody(b, _):
            off = gbase + b * 64
            pltpu.sync_copy(types_hbm.at[pl.ds(off, 64)], gidxb.at[0])
            pltpu.async_copy(emb_hbm.at[gidxb.at[0]], rowsb, sem).wait()
            pltpu.sync_copy(rowsb, embx_out.at[pl.ds(off, 64)])
            return 0

        lax.fori_loop(0, _NP // _NT // _NSC // 64, gbody, 0)

    return k(dst_pad, types_pad, emb)


def _sc_agg(table2n, src_pad, dst_pad):
    mesh = plsc.VectorSubcoreMesh(core_axis_name="c", subcore_axis_name="s")

    @functools.partial(
        pl.kernel,
        out_type=jax.ShapeDtypeStruct((_NSC, _NP, 128), jnp.float32),
        mesh=mesh,
        scratch_types=[
            pltpu.VMEM((1, _KB), jnp.int32),
            pltpu.VMEM((1, _KB), jnp.int32),
            pltpu.VMEM((_KB, 128), jnp.float32),
            pltpu.VMEM_SHARED((_ACC, 128), jnp.float32),
            pltpu.SemaphoreType.DMA,
        ],
    )
    def k(tab_hbm, src_hbm, dst_hbm, out_hbm, gidx, didx, rows, acc, sem):
        c = lax.axis_index("c")
        s = lax.axis_index("s")

        def zr(i, _):
            def zq(q, _):
                rows[i, pl.ds(q * 16, 16)] = jnp.zeros((16,), jnp.float32)
                return 0

            lax.fori_loop(0, 8, zq, 0)
            return 0

        lax.fori_loop(0, _KB, zr, 0)
        for j in range(4):
            pltpu.sync_copy(rows, acc.at[pl.ds(s * _RPT + j * _KB, _KB)])
        pltpu.sync_copy(rows.at[pl.ds(0, _RPT - 4 * _KB)],
                        acc.at[pl.ds(s * _RPT + 4 * _KB, _RPT - 4 * _KB)])
        plsc.subcore_barrier()

        base = s * _EPT

        def body(b, _):
            off = base + b * _KB
            pltpu.sync_copy(src_hbm.at[pl.ds(off, _KB)], gidx.at[0])
            pltpu.sync_copy(dst_hbm.at[pl.ds(off, _KB)], didx.at[0])
            for q in range(8):
                v = gidx[0, pl.ds(q * 16, 16)]
                gidx[0, pl.ds(q * 16, 16)] = v * 2 + c
            pltpu.async_copy(tab_hbm.at[gidx.at[0]], rows, sem).wait()
            pltpu.sync_copy(rows, acc.at[didx.at[0]], add=True)
            return 0

        lax.fori_loop(0, _STEPS, body, 0)
        plsc.subcore_barrier()
        pltpu.sync_copy(acc.at[pl.ds(s * _RPT, _RPT)],
                        out_hbm.at[c, pl.ds(s * _RPT, _RPT)])

    return k(table2n, src_pad, dst_pad)


def _tc_stage_b(x_pad, embx, deg16, w1a, w1b):
    def body(x_ref, e_ref, d_ref, wa_ref, wb_ref, o_ref):
        dinv = lax.rsqrt(d_ref[0, :, 0:1] + d_ref[1, :, 0:1] + 1.0)
        hw = jnp.dot(x_ref[...], wa_ref[...],
                     preferred_element_type=jnp.float32)
        hw = hw + jnp.dot(e_ref[:, 0:_EMB], wb_ref[...],
                          preferred_element_type=jnp.float32)
        o_ref[...] = hw * dinv

    return pl.pallas_call(
        body,
        grid=(_NP // _NB,),
        in_specs=[
            pl.BlockSpec((_NB, _F_IN), lambda i: (i, 0)),
            pl.BlockSpec((_NB, 128), lambda i: (i, 0)),
            pl.BlockSpec((_NSC, _NB, 128), lambda i: (0, i, 0)),
            pl.BlockSpec((_F_IN, _H1), lambda i: (0, 0)),
            pl.BlockSpec((_EMB, _H1), lambda i: (0, 0)),
        ],
        out_specs=pl.BlockSpec((_NB, _H1), lambda i: (i, 0)),
        out_shape=jax.ShapeDtypeStruct((_NP, _H1), jnp.float32),
    )(x_pad, embx, deg16, w1a, w1b)


def _tc_stage_d(agg1, hws1, deg16, w2, b1r, g1r, bb1r):
    def body(a_ref, h_ref, d_ref, w_ref, b_ref, g_ref, bb_ref, o_ref):
        dinv = lax.rsqrt(d_ref[0, :, 0:1] + d_ref[1, :, 0:1] + 1.0)
        cat = jnp.concatenate([a_ref[0], a_ref[1]], axis=1)
        o1 = dinv * (cat + h_ref[...]) + b_ref[...]
        mu = jnp.mean(o1, axis=1, keepdims=True)
        var = jnp.mean((o1 - mu) * (o1 - mu), axis=1, keepdims=True)
        hn = (o1 - mu) * lax.rsqrt(var + 1e-5) * g_ref[...] + bb_ref[...]
        h = jnp.maximum(hn, 0.0)
        o_ref[...] = jnp.dot(h, w_ref[...],
                             preferred_element_type=jnp.float32) * dinv

    return pl.pallas_call(
        body,
        grid=(_NP // _NB,),
        in_specs=[
            pl.BlockSpec((_NSC, _NB, 128), lambda i: (0, i, 0)),
            pl.BlockSpec((_NB, _H1), lambda i: (i, 0)),
            pl.BlockSpec((_NSC, _NB, 128), lambda i: (0, i, 0)),
            pl.BlockSpec((_H1, _H2), lambda i: (0, 0)),
            pl.BlockSpec((1, _H1), lambda i: (0, 0)),
            pl.BlockSpec((1, _H1), lambda i: (0, 0)),
            pl.BlockSpec((1, _H1), lambda i: (0, 0)),
        ],
        out_specs=pl.BlockSpec((_NB, _H2), lambda i: (i, 0)),
        out_shape=jax.ShapeDtypeStruct((_NP, _H2), jnp.float32),
    )(agg1, hws1, deg16, w2, b1r, g1r, bb1r)


def _tc_stage_f1(agg2, hws2, deg16, b2r, g2r, bb2r, gw1, gb1r, gw2p, gb2r):
    def body(a_ref, h_ref, d_ref, b_ref, g_ref, bb_ref, w1_ref, c1_ref,
             w2_ref, c2_ref, h_out, gate_out):
        dinv = lax.rsqrt(d_ref[0, :, 0:1] + d_ref[1, :, 0:1] + 1.0)
        cat = jnp.concatenate([a_ref[0], a_ref[1]], axis=1)
        o2 = dinv * (cat + h_ref[...]) + b_ref[...]
        mu = jnp.mean(o2, axis=1, keepdims=True)
        var = jnp.mean((o2 - mu) * (o2 - mu), axis=1, keepdims=True)
        hn = (o2 - mu) * lax.rsqrt(var + 1e-5) * g_ref[...] + bb_ref[...]
        h2 = jnp.maximum(hn, 0.0)
        t = jnp.maximum(
            jnp.dot(h2, w1_ref[...], preferred_element_type=jnp.float32)
            + c1_ref[...], 0.0)
        gate = jnp.dot(t, w2_ref[...],
                       preferred_element_type=jnp.float32) + c2_ref[...]
        h_out[...] = h2
        gate_out[...] = gate

    return pl.pallas_call(
        body,
        grid=(_NP // _NB,),
        in_specs=[
            pl.BlockSpec((_NSC, _NB, 128), lambda i: (0, i, 0)),
            pl.BlockSpec((_NB, _H2), lambda i: (i, 0)),
            pl.BlockSpec((_NSC, _NB, 128), lambda i: (0, i, 0)),
            pl.BlockSpec((1, _H2), lambda i: (0, 0)),
            pl.BlockSpec((1, _H2), lambda i: (0, 0)),
            pl.BlockSpec((1, _H2), lambda i: (0, 0)),
            pl.BlockSpec((_H2, _PH), lambda i: (0, 0)),
            pl.BlockSpec((1, _PH), lambda i: (0, 0)),
            pl.BlockSpec((_PH, 128), lambda i: (0, 0)),
            pl.BlockSpec((1, 1), lambda i: (0, 0)),
        ],
        out_specs=[
            pl.BlockSpec((_NB, _H2), lambda i: (i, 0)),
            pl.BlockSpec((_NB, 128), lambda i: (i, 0)),
        ],
        out_shape=[
            jax.ShapeDtypeStruct((_NP, _H2), jnp.float32),
            jax.ShapeDtypeStruct((_NP, 128), jnp.float32),
        ],
    )(agg2, hws2, deg16, b2r, g2r, bb2r, gw1, gb1r, gw2p, gb2r)


def _tc_stage_f2(h2, gate128, batch2d, gfeat, cw1a, cw1b, cb1r, cw2p, cb2r):
    def body(h_ref, g_ref, b_ref, gf_ref, w1a_ref, w1b_ref, c1_ref, w2_ref,
             c2_ref, o_ref):
        gate = g_ref[:, 0:1]
        bb = b_ref[...]
        seg = lax.broadcasted_iota(jnp.int32, (_NP, _NG), 1)
        obool = bb == seg
        onehot = obool.astype(jnp.float32)
        masked = jnp.where(obool, gate, -jnp.inf)
        m = jnp.max(masked, axis=0, keepdims=True)
        m = jnp.where(m == -jnp.inf, 0.0, m)
        mrow = jnp.sum(onehot * m, axis=1, keepdims=True)
        valid = bb < _NG
        e = jnp.where(valid, jnp.exp(gate - mrow), 0.0)
        ssum = jnp.sum(onehot * e, axis=0, keepdims=True)
        srow = jnp.sum(onehot * ssum, axis=1, keepdims=True)
        alpha = e / (srow + 1e-16)
        ha = jnp.where(valid, h_ref[...] * alpha, 0.0)
        pooled = lax.dot_general(onehot, ha, (((0,), (0,)), ((), ())),
                                 preferred_element_type=jnp.float32)
        t = jnp.maximum(
            jnp.dot(pooled, w1a_ref[...], preferred_element_type=jnp.float32)
            + jnp.dot(gf_ref[...], w1b_ref[...],
                      preferred_element_type=jnp.float32)
            + c1_ref[...], 0.0)
        o = jnp.dot(t, w2_ref[...], preferred_element_type=jnp.float32)
        o_ref[...] = o[:, 0:2] + c2_ref[...]

    return pl.pallas_call(
        body,
        in_specs=[
            pl.BlockSpec((_NP, _H2), lambda: (0, 0)),
            pl.BlockSpec((_NP, 128), lambda: (0, 0)),
            pl.BlockSpec((_NP, 1), lambda: (0, 0)),
            pl.BlockSpec((_NG, _GDIM), lambda: (0, 0)),
            pl.BlockSpec((_H2, _PH), lambda: (0, 0)),
            pl.BlockSpec((_GDIM, _PH), lambda: (0, 0)),
            pl.BlockSpec((1, _PH), lambda: (0, 0)),
            pl.BlockSpec((_PH, 128), lambda: (0, 0)),
            pl.BlockSpec((1, 2), lambda: (0, 0)),
        ],
        out_specs=pl.BlockSpec((_NG, 2), lambda: (0, 0)),
        out_shape=jax.ShapeDtypeStruct((_NG, 2), jnp.float32),
    )(h2, gate128, batch2d, gfeat, cw1a, cw1b, cb1r, cw2p, cb2r)


def kernel(x, edge_index, batch, global_features, emb, W1, b1, ln1_g, ln1_b,
           W2, b2, ln2_g, ln2_b, gW1, gb1, gW2, gb2, cW1, cb1, cW2, cb2):
    f32 = jnp.float32
    node_types = x[:, 0].astype(jnp.int32)
    types_pad = jnp.concatenate(
        [node_types, jnp.zeros((_NP - _N,), jnp.int32)])
    x_pad = jnp.concatenate([x, jnp.zeros((_NP - _N, _F_IN), f32)])
    padi = jnp.arange(_EP - _E, dtype=jnp.int32)
    src_pad = jnp.concatenate([edge_index[0], padi % 128])
    dst_pad = jnp.concatenate([edge_index[1], _N + padi % (_ACC - _N)])
    batch2d = jnp.concatenate(
        [batch, jnp.full((_NP - _N,), _NG, jnp.int32)]).reshape(_NP, 1)

    w1a = jnp.concatenate([jnp.zeros((1, _H1), f32), W1[:_F_IN - 1]], axis=0)
    w1b = W1[_F_IN - 1:]
    b1r = b1.reshape(1, _H1)
    g1r = ln1_g.reshape(1, _H1)
    bb1r = ln1_b.reshape(1, _H1)
    b2r = b2.reshape(1, _H2)
    g2r = ln2_g.reshape(1, _H2)
    bb2r = ln2_b.reshape(1, _H2)
    gb1r = gb1.reshape(1, _PH)
    gw2p = jnp.concatenate([gW2, jnp.zeros((_PH, 127), f32)], axis=1)
    gb2r = gb2.reshape(1, 1)
    cw1a = cW1[:_H2]
    cw1b = cW1[_H2:]
    cb1r = cb1.reshape(1, _PH)
    cw2p = jnp.concatenate([cW2, jnp.zeros((_PH, 126), f32)], axis=1)
    cb2r = cb2.reshape(1, 2)

    emb_pad = jnp.concatenate(
        [emb, jnp.zeros((emb.shape[0], 128 - _EMB), f32)], axis=1)
    degw, embx = _sc_deg_emb(dst_pad, types_pad, emb_pad)

    hws1 = _tc_stage_b(x_pad, embx, degw, w1a, w1b)

    agg1 = _sc_agg(hws1.reshape(_NP * 2, 128), src_pad, dst_pad)
    hws2 = _tc_stage_d(agg1, hws1, degw, W2, b1r, g1r, bb1r)
    agg2 = _sc_agg(hws2.reshape(_NP * 2, 128), src_pad, dst_pad)
    h2, gate128 = _tc_stage_f1(agg2, hws2, degw, b2r, g2r, bb2r,
                               gW1, gb1r, gw2p, gb2r)
    return _tc_stage_f2(h2, gate128, batch2d, global_features,
                        cw1a, cw1b, cb1r, cw2p, cb2r)


# speedup vs baseline: 11.0981x; 1.5072x over previous
import functools

import jax
import jax.numpy as jnp
from jax import lax
from jax.experimental import pallas as pl
from jax.experimental.pallas import tpu as pltpu
from jax.experimental.pallas import tpu_sc as plsc

_N = 10000
_E = 320000
_F_IN = 128
_EMB = 16
_GDIM = 16
_NG = 64
_H1 = 256
_H2 = 256
_PH = 128

_NP = 10240
_NB = 512
_NSC = 2
_NT = 16
_KB = 128
_GRP = 16
_EP = 327680
_STEPS = _EP // _NT // _KB
_AGRP = _STEPS // _GRP
_DSTEPS = _EP // (_NT * _NSC) // _KB
_DGRP = _DSTEPS // _GRP
_EPT = _KB * _STEPS
_ACC = 10112
_RPT = _ACC // _NT


def _sc_deg_emb(dst_pad, types_pad, emb):
    mesh = plsc.VectorSubcoreMesh(core_axis_name="c", subcore_axis_name="s")

    @functools.partial(
        pl.kernel,
        out_type=[
            jax.ShapeDtypeStruct((_NSC, _NP, 128), jnp.float32),
            jax.ShapeDtypeStruct((_NP, 128), jnp.float32),
        ],
        mesh=mesh,
        scratch_types=[
            pltpu.VMEM((_GRP, _KB), jnp.int32),
            pltpu.VMEM((1, 64), jnp.int32),
            pltpu.VMEM((_KB, 128), jnp.float32),
            pltpu.VMEM((64, 128), jnp.float32),
            pltpu.VMEM_SHARED((_ACC, 128), jnp.float32),
            pltpu.SemaphoreType.DMA,
            pltpu.SemaphoreType.DMA,
        ],
    )
    def k(dst_hbm, types_hbm, emb_hbm, deg_out, embx_out, idxb, gidxb,
          onesb, rowsb, acc, sem, ssem):
        c = lax.axis_index("c")
        s = lax.axis_index("s")
        wid = s * _NSC + c

        def zr(i, _):
            def zq(q, _):
                onesb[i, pl.ds(q * 16, 16)] = jnp.zeros((16,), jnp.float32)
                return 0

            lax.fori_loop(0, 8, zq, 0)
            return 0

        lax.fori_loop(0, _KB, zr, 0)
        for j in range(4):
            pltpu.sync_copy(onesb, acc.at[pl.ds(s * _RPT + j * _KB, _KB)])
        pltpu.sync_copy(onesb.at[pl.ds(0, _RPT - 4 * _KB)],
                        acc.at[pl.ds(s * _RPT + 4 * _KB, _RPT - 4 * _KB)])

        def orow(i, _):
            def oq(q, _):
                onesb[i, pl.ds(q * 16, 16)] = jnp.ones((16,), jnp.float32)
                return 0

            lax.fori_loop(0, 8, oq, 0)
            return 0

        lax.fori_loop(0, _KB, orow, 0)
        plsc.subcore_barrier()

        drow = wid * _DSTEPS

        def body(g, _):
            pltpu.sync_copy(dst_hbm.at[pl.ds(drow + g * _GRP, _GRP)], idxb)
            descs = []
            for j in range(_GRP):
                descs.append(pltpu.async_copy(
                    onesb, acc.at[idxb.at[j]], ssem, add=True))
            for d in descs:
                d.wait()
            return 0

        lax.fori_loop(0, _DGRP, body, 0)
        plsc.subcore_barrier()
        pltpu.sync_copy(acc.at[pl.ds(s * _RPT, _RPT)],
                        deg_out.at[c, pl.ds(s * _RPT, _RPT)])

        gbase = wid * (_NP // _NT // _NSC)

        def gbody(b, _):
            off = gbase + b * 64
            pltpu.sync_copy(types_hbm.at[pl.ds(off, 64)], gidxb.at[0])
            pltpu.async_copy(emb_hbm.at[gidxb.at[0]], rowsb, sem).wait()
            pltpu.sync_copy(rowsb, embx_out.at[pl.ds(off, 64)])
            return 0

        lax.fori_loop(0, _NP // _NT // _NSC // 64, gbody, 0)

    return k(dst_pad, types_pad, emb)


def _sc_agg(table2n, src_pad, dst_pad):
    mesh = plsc.VectorSubcoreMesh(core_axis_name="c", subcore_axis_name="s")

    @functools.partial(
        pl.kernel,
        out_type=jax.ShapeDtypeStruct((_NSC, _NP, 128), jnp.float32),
        mesh=mesh,
        scratch_types=[
            pltpu.VMEM((_GRP, _KB), jnp.int32),
            pltpu.VMEM((_GRP, _KB), jnp.int32),
            pltpu.VMEM((2, _KB, 128), jnp.float32),
            pltpu.VMEM_SHARED((_ACC, 128), jnp.float32),
            pltpu.SemaphoreType.DMA,
            pltpu.SemaphoreType.DMA,
        ],
    )
    def k(tab_hbm, src_hbm, dst_hbm, out_hbm, gidx, didx, rows, acc, sem,
          sem2):
        c = lax.axis_index("c")
        s = lax.axis_index("s")

        def zr(i, _):
            def zq(q, _):
                rows[0, i, pl.ds(q * 16, 16)] = jnp.zeros((16,), jnp.float32)
                return 0

            lax.fori_loop(0, 8, zq, 0)
            return 0

        lax.fori_loop(0, _KB, zr, 0)
        for j in range(4):
            pltpu.sync_copy(rows.at[0],
                            acc.at[pl.ds(s * _RPT + j * _KB, _KB)])
        pltpu.sync_copy(rows.at[0, pl.ds(0, _RPT - 4 * _KB)],
                        acc.at[pl.ds(s * _RPT + 4 * _KB, _RPT - 4 * _KB)])
        plsc.subcore_barrier()

        grow = s * _STEPS

        def xform(j):
            def xq(q, _):
                v = gidx[j, pl.ds(q * 16, 16)]
                gidx[j, pl.ds(q * 16, 16)] = v * 2 + c
                return 0

            lax.fori_loop(0, 8, xq, 0)

        def body(g, _):
            pltpu.sync_copy(src_hbm.at[pl.ds(grow + g * _GRP, _GRP)], gidx)
            pltpu.sync_copy(dst_hbm.at[pl.ds(grow + g * _GRP, _GRP)], didx)
            xform(0)
            desc = pltpu.async_copy(tab_hbm.at[gidx.at[0]], rows.at[0], sem)
            for j in range(_GRP):
                if j + 1 < _GRP:
                    xform(j + 1)
                    nxt = pltpu.async_copy(
                        tab_hbm.at[gidx.at[j + 1]],
                        rows.at[(j + 1) % 2],
                        sem2 if (j + 1) % 2 else sem)
                desc.wait()
                pltpu.sync_copy(rows.at[j % 2], acc.at[didx.at[j]],
                                add=True)
                if j + 1 < _GRP:
                    desc = nxt
            return 0

        lax.fori_loop(0, _AGRP, body, 0)
        plsc.subcore_barrier()
        pltpu.sync_copy(acc.at[pl.ds(s * _RPT, _RPT)],
                        out_hbm.at[c, pl.ds(s * _RPT, _RPT)])

    return k(table2n, src_pad, dst_pad)


def _tc_stage_b(x_pad, embx, deg16, w1a, w1b):
    def body(x_ref, e_ref, d_ref, wa_ref, wb_ref, o_ref):
        dinv = lax.rsqrt(d_ref[0, :, 0:1] + d_ref[1, :, 0:1] + 1.0)
        hw = jnp.dot(x_ref[...], wa_ref[...],
                     preferred_element_type=jnp.float32)
        hw = hw + jnp.dot(e_ref[:, 0:_EMB], wb_ref[...],
                          preferred_element_type=jnp.float32)
        o_ref[...] = hw * dinv

    return pl.pallas_call(
        body,
        grid=(_NP // _NB,),
        in_specs=[
            pl.BlockSpec((_NB, _F_IN), lambda i: (i, 0)),
            pl.BlockSpec((_NB, 128), lambda i: (i, 0)),
            pl.BlockSpec((_NSC, _NB, 128), lambda i: (0, i, 0)),
            pl.BlockSpec((_F_IN, _H1), lambda i: (0, 0)),
            pl.BlockSpec((_EMB, _H1), lambda i: (0, 0)),
        ],
        out_specs=pl.BlockSpec((_NB, _H1), lambda i: (i, 0)),
        out_shape=jax.ShapeDtypeStruct((_NP, _H1), jnp.float32),
    )(x_pad, embx, deg16, w1a, w1b)


def _tc_stage_d(agg1, hws1, deg16, w2, b1r, g1r, bb1r):
    def body(a_ref, h_ref, d_ref, w_ref, b_ref, g_ref, bb_ref, o_ref):
        dinv = lax.rsqrt(d_ref[0, :, 0:1] + d_ref[1, :, 0:1] + 1.0)
        cat = jnp.concatenate([a_ref[0], a_ref[1]], axis=1)
        o1 = dinv * (cat + h_ref[...]) + b_ref[...]
        mu = jnp.mean(o1, axis=1, keepdims=True)
        var = jnp.mean((o1 - mu) * (o1 - mu), axis=1, keepdims=True)
        hn = (o1 - mu) * lax.rsqrt(var + 1e-5) * g_ref[...] + bb_ref[...]
        h = jnp.maximum(hn, 0.0)
        o_ref[...] = jnp.dot(h, w_ref[...],
                             preferred_element_type=jnp.float32) * dinv

    return pl.pallas_call(
        body,
        grid=(_NP // _NB,),
        in_specs=[
            pl.BlockSpec((_NSC, _NB, 128), lambda i: (0, i, 0)),
            pl.BlockSpec((_NB, _H1), lambda i: (i, 0)),
            pl.BlockSpec((_NSC, _NB, 128), lambda i: (0, i, 0)),
            pl.BlockSpec((_H1, _H2), lambda i: (0, 0)),
            pl.BlockSpec((1, _H1), lambda i: (0, 0)),
            pl.BlockSpec((1, _H1), lambda i: (0, 0)),
            pl.BlockSpec((1, _H1), lambda i: (0, 0)),
        ],
        out_specs=pl.BlockSpec((_NB, _H2), lambda i: (i, 0)),
        out_shape=jax.ShapeDtypeStruct((_NP, _H2), jnp.float32),
    )(agg1, hws1, deg16, w2, b1r, g1r, bb1r)


def _tc_stage_f1(agg2, hws2, deg16, b2r, g2r, bb2r, gw1, gb1r, gw2p, gb2r):
    def body(a_ref, h_ref, d_ref, b_ref, g_ref, bb_ref, w1_ref, c1_ref,
             w2_ref, c2_ref, h_out, gate_out):
        dinv = lax.rsqrt(d_ref[0, :, 0:1] + d_ref[1, :, 0:1] + 1.0)
        cat = jnp.concatenate([a_ref[0], a_ref[1]], axis=1)
        o2 = dinv * (cat + h_ref[...]) + b_ref[...]
        mu = jnp.mean(o2, axis=1, keepdims=True)
        var = jnp.mean((o2 - mu) * (o2 - mu), axis=1, keepdims=True)
        hn = (o2 - mu) * lax.rsqrt(var + 1e-5) * g_ref[...] + bb_ref[...]
        h2 = jnp.maximum(hn, 0.0)
        t = jnp.maximum(
            jnp.dot(h2, w1_ref[...], preferred_element_type=jnp.float32)
            + c1_ref[...], 0.0)
        gate = jnp.dot(t, w2_ref[...],
                       preferred_element_type=jnp.float32) + c2_ref[...]
        h_out[...] = h2
        gate_out[...] = gate

    return pl.pallas_call(
        body,
        grid=(_NP // _NB,),
        in_specs=[
            pl.BlockSpec((_NSC, _NB, 128), lambda i: (0, i, 0)),
            pl.BlockSpec((_NB, _H2), lambda i: (i, 0)),
            pl.BlockSpec((_NSC, _NB, 128), lambda i: (0, i, 0)),
            pl.BlockSpec((1, _H2), lambda i: (0, 0)),
            pl.BlockSpec((1, _H2), lambda i: (0, 0)),
            pl.BlockSpec((1, _H2), lambda i: (0, 0)),
            pl.BlockSpec((_H2, _PH), lambda i: (0, 0)),
            pl.BlockSpec((1, _PH), lambda i: (0, 0)),
            pl.BlockSpec((_PH, 128), lambda i: (0, 0)),
            pl.BlockSpec((1, 1), lambda i: (0, 0)),
        ],
        out_specs=[
            pl.BlockSpec((_NB, _H2), lambda i: (i, 0)),
            pl.BlockSpec((_NB, 128), lambda i: (i, 0)),
        ],
        out_shape=[
            jax.ShapeDtypeStruct((_NP, _H2), jnp.float32),
            jax.ShapeDtypeStruct((_NP, 128), jnp.float32),
        ],
    )(agg2, hws2, deg16, b2r, g2r, bb2r, gw1, gb1r, gw2p, gb2r)


def _tc_stage_f2(h2, gate128, batch2d, gfeat, cw1a, cw1b, cb1r, cw2p, cb2r):
    def body(h_ref, g_ref, b_ref, gf_ref, w1a_ref, w1b_ref, c1_ref, w2_ref,
             c2_ref, o_ref):
        gate = g_ref[:, 0:1]
        bb = b_ref[...]
        seg = lax.broadcasted_iota(jnp.int32, (_NP, _NG), 1)
        obool = bb == seg
        onehot = obool.astype(jnp.float32)
        masked = jnp.where(obool, gate, -jnp.inf)
        m = jnp.max(masked, axis=0, keepdims=True)
        m = jnp.where(m == -jnp.inf, 0.0, m)
        mrow = jnp.sum(onehot * m, axis=1, keepdims=True)
        valid = bb < _NG
        e = jnp.where(valid, jnp.exp(gate - mrow), 0.0)
        ssum = jnp.sum(onehot * e, axis=0, keepdims=True)
        srow = jnp.sum(onehot * ssum, axis=1, keepdims=True)
        alpha = e / (srow + 1e-16)
        ha = jnp.where(valid, h_ref[...] * alpha, 0.0)
        pooled = lax.dot_general(onehot, ha, (((0,), (0,)), ((), ())),
                                 preferred_element_type=jnp.float32)
        t = jnp.maximum(
            jnp.dot(pooled, w1a_ref[...], preferred_element_type=jnp.float32)
            + jnp.dot(gf_ref[...], w1b_ref[...],
                      preferred_element_type=jnp.float32)
            + c1_ref[...], 0.0)
        o = jnp.dot(t, w2_ref[...], preferred_element_type=jnp.float32)
        o_ref[...] = o[:, 0:2] + c2_ref[...]

    return pl.pallas_call(
        body,
        in_specs=[
            pl.BlockSpec((_NP, _H2), lambda: (0, 0)),
            pl.BlockSpec((_NP, 128), lambda: (0, 0)),
            pl.BlockSpec((_NP, 1), lambda: (0, 0)),
            pl.BlockSpec((_NG, _GDIM), lambda: (0, 0)),
            pl.BlockSpec((_H2, _PH), lambda: (0, 0)),
            pl.BlockSpec((_GDIM, _PH), lambda: (0, 0)),
            pl.BlockSpec((1, _PH), lambda: (0, 0)),
            pl.BlockSpec((_PH, 128), lambda: (0, 0)),
            pl.BlockSpec((1, 2), lambda: (0, 0)),
        ],
        out_specs=pl.BlockSpec((_NG, 2), lambda: (0, 0)),
        out_shape=jax.ShapeDtypeStruct((_NG, 2), jnp.float32),
    )(h2, gate128, batch2d, gfeat, cw1a, cw1b, cb1r, cw2p, cb2r)


def kernel(x, edge_index, batch, global_features, emb, W1, b1, ln1_g, ln1_b,
           W2, b2, ln2_g, ln2_b, gW1, gb1, gW2, gb2, cW1, cb1, cW2, cb2):
    f32 = jnp.float32
    node_types = x[:, 0].astype(jnp.int32)
    types_pad = jnp.concatenate(
        [node_types, jnp.zeros((_NP - _N,), jnp.int32)])
    x_pad = jnp.concatenate([x, jnp.zeros((_NP - _N, _F_IN), f32)])
    padi = jnp.arange(_EP - _E, dtype=jnp.int32)
    src_pad = jnp.concatenate(
        [edge_index[0], padi % 128]).reshape(_EP // _KB, _KB)
    dst_pad = jnp.concatenate(
        [edge_index[1], _N + padi % (_ACC - _N)]).reshape(_EP // _KB, _KB)
    batch2d = jnp.concatenate(
        [batch, jnp.full((_NP - _N,), _NG, jnp.int32)]).reshape(_NP, 1)

    w1a = jnp.concatenate([jnp.zeros((1, _H1), f32), W1[:_F_IN - 1]], axis=0)
    w1b = W1[_F_IN - 1:]
    b1r = b1.reshape(1, _H1)
    g1r = ln1_g.reshape(1, _H1)
    bb1r = ln1_b.reshape(1, _H1)
    b2r = b2.reshape(1, _H2)
    g2r = ln2_g.reshape(1, _H2)
    bb2r = ln2_b.reshape(1, _H2)
    gb1r = gb1.reshape(1, _PH)
    gw2p = jnp.concatenate([gW2, jnp.zeros((_PH, 127), f32)], axis=1)
    gb2r = gb2.reshape(1, 1)
    cw1a = cW1[:_H2]
    cw1b = cW1[_H2:]
    cb1r = cb1.reshape(1, _PH)
    cw2p = jnp.concatenate([cW2, jnp.zeros((_PH, 126), f32)], axis=1)
    cb2r = cb2.reshape(1, 2)

    emb_pad = jnp.concatenate(
        [emb, jnp.zeros((emb.shape[0], 128 - _EMB), f32)], axis=1)
    degw, embx = _sc_deg_emb(dst_pad, types_pad, emb_pad)

    hws1 = _tc_stage_b(x_pad, embx, degw, w1a, w1b)

    agg1 = _sc_agg(hws1.reshape(_NP * 2, 128), src_pad, dst_pad)
    hws2 = _tc_stage_d(agg1, hws1, degw, W2, b1r, g1r, bb1r)
    agg2 = _sc_agg(hws2.reshape(_NP * 2, 128), src_pad, dst_pad)
    h2, gate128 = _tc_stage_f1(agg2, hws2, degw, b2r, g2r, bb2r,
                               gW1, gb1r, gw2p, gb2r)
    return _tc_stage_f2(h2, gate128, batch2d, global_features,
                        cw1a, cw1b, cb1r, cw2p, cb2r)
